# baseline (speedup 1.0000x reference)
import jax
import jax.numpy as jnp
from jax.experimental import pallas as pl
from jax.experimental.pallas import tpu as pltpu

_N = 100
_M = 28
_H = 200
_W = 320
_STUFF_C = 53
_STRIDE = 4
_K = 3


def _paste_one(ibox_ref, fwh_ref, m, sem, n):
    a = ibox_ref[n, 0]
    b = ibox_ref[n, 1]
    c = ibox_ref[n, 2]
    d = ibox_ref[n, 3]
    cx1 = ibox_ref[n, 4]
    cy1 = ibox_ref[n, 5]
    cx2 = ibox_ref[n, 6]
    cy2 = ibox_ref[n, 7]
    ww = fwh_ref[n, 0]
    hh = fwh_ref[n, 1]

    jj = jax.lax.broadcasted_iota(jnp.int32, (_M, _H), 0)
    ys = jax.lax.broadcasted_iota(jnp.int32, (_M, _H), 1)
    v = (ys.astype(jnp.float32) - b.astype(jnp.float32) + 0.5) * (
        jnp.float32(_M) / hh) - 0.5
    v = jnp.clip(v, 0.0, jnp.float32(_M - 1))
    v0 = jnp.floor(v).astype(jnp.int32)
    v1 = jnp.minimum(v0 + 1, _M - 1)
    fv = v - v0.astype(jnp.float32)
    vy = (ys >= jnp.maximum(b, 0)) & (ys < jnp.minimum(d + 1, _H))
    r_vt = ((jj == v0).astype(jnp.float32) * (1.0 - fv)
            + (jj == v1).astype(jnp.float32) * fv) * vy.astype(jnp.float32)

    kk = jax.lax.broadcasted_iota(jnp.int32, (_M, _W), 0)
    xs = jax.lax.broadcasted_iota(jnp.int32, (_M, _W), 1)
    u = (xs.astype(jnp.float32) - a.astype(jnp.float32) + 0.5) * (
        jnp.float32(_M) / ww) - 0.5
    u = jnp.clip(u, 0.0, jnp.float32(_M - 1))
    u0 = jnp.floor(u).astype(jnp.int32)
    u1 = jnp.minimum(u0 + 1, _M - 1)
    fu = u - u0.astype(jnp.float32)
    vx = (xs >= jnp.maximum(a, 0)) & (xs < jnp.minimum(c + 1, _W))
    r_u = ((kk == u0).astype(jnp.float32) * (1.0 - fu)
           + (kk == u1).astype(jnp.float32) * fu) * vx.astype(jnp.float32)

    tm = jax.lax.dot_general(
        r_vt, m, dimension_numbers=(((0,), (0,)), ((), ())),
        preferred_element_type=jnp.float32,
        precision=jax.lax.Precision.DEFAULT)
    tm = jnp.dot(tm, r_u, preferred_element_type=jnp.float32,
                 precision=jax.lax.Precision.DEFAULT)

    ys2 = jax.lax.broadcasted_iota(jnp.int32, (_H, 1), 0)
    xs2 = jax.lax.broadcasted_iota(jnp.int32, (1, _W), 1)
    rowm = ((ys2 >= cy1) & (ys2 < cy2)).astype(jnp.float32)
    colm = ((xs2 >= cx1) & (xs2 < cx2)).astype(jnp.float32)
    return tm + sem * rowm * colm


def _pan_kernel(sem_idx_ref, ibox_ref, fwh_ref,
                m0_ref, m1_ref, m2_ref, s0_ref, s1_ref, s2_ref, out_ref):
    i = pl.program_id(0)
    for j, (m_ref, s_ref) in enumerate(
            ((m0_ref, s0_ref), (m1_ref, s1_ref), (m2_ref, s2_ref))):
        ch = i * _K + j

        @pl.when(ch < _STUFF_C)
        def _copy_stuff(s_ref=s_ref, j=j):
            out_ref[j] = s_ref[0]

        @pl.when(ch >= _STUFF_C)
        def _paste(m_ref=m_ref, s_ref=s_ref, j=j, ch=ch):
            out_ref[j] = _paste_one(ibox_ref, fwh_ref, m_ref[0], s_ref[0],
                                    ch - _STUFF_C)


def _mask_spec(j):
    return pl.BlockSpec(
        (1, _M, _M),
        lambda i, sem_idx, ibox, fwh:
        (jnp.clip(i * _K + j - _STUFF_C, 0, _N - 1), 0, 0))


def _sem_spec(j):
    return pl.BlockSpec(
        (1, _H, _W),
        lambda i, sem_idx, ibox, fwh: (sem_idx[i * _K + j], 0, 0))


def kernel(mask_logit, sem_seg_logits, boxes, classes):
    bf = boxes / float(_STRIDE)
    bb = jnp.floor(bf).astype(jnp.int32)
    x0, y0, x2, y2 = bb[:, 0], bb[:, 1], bb[:, 2], bb[:, 3]
    w = (x2 - x0 + 1).astype(jnp.float32)
    h = (y2 - y0 + 1).astype(jnp.float32)
    cx1 = jnp.floor(bf[:, 0]).astype(jnp.int32)
    cy1 = jnp.floor(bf[:, 1]).astype(jnp.int32)
    cx2 = (jnp.round(bf[:, 2]) + 1.0).astype(jnp.int32)
    cy2 = (jnp.round(bf[:, 3]) + 1.0).astype(jnp.int32)
    ibox = jnp.stack([x0, y0, x2, y2, cx1, cy1, cx2, cy2], axis=1)
    fwh = jnp.stack([w, h], axis=1)
    sem_idx = jnp.concatenate([
        jnp.arange(_STUFF_C, dtype=jnp.int32),
        _STUFF_C + classes.astype(jnp.int32),
    ])

    grid_spec = pltpu.PrefetchScalarGridSpec(
        num_scalar_prefetch=3,
        grid=((_STUFF_C + _N) // _K,),
        in_specs=[_mask_spec(0), _mask_spec(1), _mask_spec(2),
                  _sem_spec(0), _sem_spec(1), _sem_spec(2)],
        out_specs=pl.BlockSpec((_K, _H, _W),
                               lambda i, sem_idx, ibox, fwh: (i, 0, 0)),
    )
    out = pl.pallas_call(
        _pan_kernel,
        grid_spec=grid_spec,
        out_shape=jax.ShapeDtypeStruct((_STUFF_C + _N, _H, _W), jnp.float32),
        compiler_params=pltpu.CompilerParams(
            dimension_semantics=("arbitrary",)),
    )(sem_idx, ibox, fwh, mask_logit, mask_logit, mask_logit,
      sem_seg_logits, sem_seg_logits, sem_seg_logits)
    return out[None]


# speedup vs baseline: 1.1115x; 1.1115x over previous
import jax
import jax.numpy as jnp
from jax.experimental import pallas as pl
from jax.experimental.pallas import tpu as pltpu

_N = 100
_M = 28
_H = 200
_W = 320
_STUFF_C = 53
_STRIDE = 4
_K = 9


def _paste_one(ibox_ref, fwh_ref, m, sem, n):
    a = ibox_ref[n, 0]
    b = ibox_ref[n, 1]
    c = ibox_ref[n, 2]
    d = ibox_ref[n, 3]
    cx1 = ibox_ref[n, 4]
    cy1 = ibox_ref[n, 5]
    cx2 = ibox_ref[n, 6]
    cy2 = ibox_ref[n, 7]
    ww = fwh_ref[n, 0]
    hh = fwh_ref[n, 1]

    jj = jax.lax.broadcasted_iota(jnp.int32, (_M, _H), 0)
    ys = jax.lax.broadcasted_iota(jnp.int32, (_M, _H), 1)
    v = (ys.astype(jnp.float32) - b.astype(jnp.float32) + 0.5) * (
        jnp.float32(_M) / hh) - 0.5
    v = jnp.clip(v, 0.0, jnp.float32(_M - 1))
    v0 = jnp.floor(v).astype(jnp.int32)
    v1 = jnp.minimum(v0 + 1, _M - 1)
    fv = v - v0.astype(jnp.float32)
    vy = (ys >= jnp.maximum(b, 0)) & (ys < jnp.minimum(d + 1, _H))
    r_vt = ((jj == v0).astype(jnp.float32) * (1.0 - fv)
            + (jj == v1).astype(jnp.float32) * fv) * vy.astype(jnp.float32)

    kk = jax.lax.broadcasted_iota(jnp.int32, (_M, _W), 0)
    xs = jax.lax.broadcasted_iota(jnp.int32, (_M, _W), 1)
    u = (xs.astype(jnp.float32) - a.astype(jnp.float32) + 0.5) * (
        jnp.float32(_M) / ww) - 0.5
    u = jnp.clip(u, 0.0, jnp.float32(_M - 1))
    u0 = jnp.floor(u).astype(jnp.int32)
    u1 = jnp.minimum(u0 + 1, _M - 1)
    fu = u - u0.astype(jnp.float32)
    vx = (xs >= jnp.maximum(a, 0)) & (xs < jnp.minimum(c + 1, _W))
    r_u = ((kk == u0).astype(jnp.float32) * (1.0 - fu)
           + (kk == u1).astype(jnp.float32) * fu) * vx.astype(jnp.float32)

    tm = jax.lax.dot_general(
        r_vt, m, dimension_numbers=(((0,), (0,)), ((), ())),
        preferred_element_type=jnp.float32,
        precision=jax.lax.Precision.DEFAULT)
    tm = jnp.dot(tm, r_u, preferred_element_type=jnp.float32,
                 precision=jax.lax.Precision.DEFAULT)

    ys2 = jax.lax.broadcasted_iota(jnp.int32, (_H, 1), 0)
    xs2 = jax.lax.broadcasted_iota(jnp.int32, (1, _W), 1)
    rowm = ((ys2 >= cy1) & (ys2 < cy2)).astype(jnp.float32)
    colm = ((xs2 >= cx1) & (xs2 < cx2)).astype(jnp.float32)
    return tm + sem * rowm * colm


def _pan_kernel(*refs):
    sem_idx_ref, ibox_ref, fwh_ref = refs[:3]
    m_refs = refs[3:3 + _K]
    s_refs = refs[3 + _K:3 + 2 * _K]
    out_ref = refs[3 + 2 * _K]
    i = pl.program_id(0)
    for j, (m_ref, s_ref) in enumerate(zip(m_refs, s_refs)):
        ch = i * _K + j

        @pl.when(ch < _STUFF_C)
        def _copy_stuff(s_ref=s_ref, j=j):
            out_ref[j] = s_ref[0]

        @pl.when(ch >= _STUFF_C)
        def _paste(m_ref=m_ref, s_ref=s_ref, j=j, ch=ch):
            out_ref[j] = _paste_one(ibox_ref, fwh_ref, m_ref[0], s_ref[0],
                                    ch - _STUFF_C)


def _mask_spec(j):
    return pl.BlockSpec(
        (1, _M, _M),
        lambda i, sem_idx, ibox, fwh:
        (jnp.clip(i * _K + j - _STUFF_C, 0, _N - 1), 0, 0))


def _sem_spec(j):
    return pl.BlockSpec(
        (1, _H, _W),
        lambda i, sem_idx, ibox, fwh: (sem_idx[i * _K + j], 0, 0))


def kernel(mask_logit, sem_seg_logits, boxes, classes):
    bf = boxes / float(_STRIDE)
    bb = jnp.floor(bf).astype(jnp.int32)
    x0, y0, x2, y2 = bb[:, 0], bb[:, 1], bb[:, 2], bb[:, 3]
    w = (x2 - x0 + 1).astype(jnp.float32)
    h = (y2 - y0 + 1).astype(jnp.float32)
    cx1 = jnp.floor(bf[:, 0]).astype(jnp.int32)
    cy1 = jnp.floor(bf[:, 1]).astype(jnp.int32)
    cx2 = (jnp.round(bf[:, 2]) + 1.0).astype(jnp.int32)
    cy2 = (jnp.round(bf[:, 3]) + 1.0).astype(jnp.int32)
    ibox = jnp.stack([x0, y0, x2, y2, cx1, cy1, cx2, cy2], axis=1)
    fwh = jnp.stack([w, h], axis=1)
    sem_idx = jnp.concatenate([
        jnp.arange(_STUFF_C, dtype=jnp.int32),
        _STUFF_C + classes.astype(jnp.int32),
    ])

    grid_spec = pltpu.PrefetchScalarGridSpec(
        num_scalar_prefetch=3,
        grid=((_STUFF_C + _N) // _K,),
        in_specs=([_mask_spec(j) for j in range(_K)]
                  + [_sem_spec(j) for j in range(_K)]),
        out_specs=pl.BlockSpec((_K, _H, _W),
                               lambda i, sem_idx, ibox, fwh: (i, 0, 0)),
    )
    out = pl.pallas_call(
        _pan_kernel,
        grid_spec=grid_spec,
        out_shape=jax.ShapeDtypeStruct((_STUFF_C + _N, _H, _W), jnp.float32),
        compiler_params=pltpu.CompilerParams(
            dimension_semantics=("arbitrary",)),
    )(sem_idx, ibox, fwh, *([mask_logit] * _K), *([sem_seg_logits] * _K))
    return out[None]


# speedup vs baseline: 1.3727x; 1.2350x over previous
import jax
import jax.numpy as jnp
from jax.experimental import pallas as pl
from jax.experimental.pallas import tpu as pltpu

_N = 100
_M = 28
_H = 200
_W = 320
_STUFF_C = 53
_STRIDE = 4
_K = 9


def _paste_one(ibox_ref, fwh_ref, m, sem, n):
    a = ibox_ref[n, 0]
    b = ibox_ref[n, 1]
    c = ibox_ref[n, 2]
    d = ibox_ref[n, 3]
    cx1 = ibox_ref[n, 4]
    cy1 = ibox_ref[n, 5]
    cx2 = ibox_ref[n, 6]
    cy2 = ibox_ref[n, 7]
    ww = fwh_ref[n, 0]
    hh = fwh_ref[n, 1]

    jj = jax.lax.broadcasted_iota(jnp.int32, (_M, _H), 0)
    ys = jax.lax.broadcasted_iota(jnp.int32, (_M, _H), 1)
    v = (ys.astype(jnp.float32) - b.astype(jnp.float32) + 0.5) * (
        jnp.float32(_M) / hh) - 0.5
    v = jnp.clip(v, 0.0, jnp.float32(_M - 1))
    v0 = jnp.floor(v).astype(jnp.int32)
    v1 = jnp.minimum(v0 + 1, _M - 1)
    fv = v - v0.astype(jnp.float32)
    vy = (ys >= jnp.maximum(b, 0)) & (ys < jnp.minimum(d + 1, _H))
    r_vt = ((jj == v0).astype(jnp.float32) * (1.0 - fv)
            + (jj == v1).astype(jnp.float32) * fv) * vy.astype(jnp.float32)

    kk = jax.lax.broadcasted_iota(jnp.int32, (_M, _W), 0)
    xs = jax.lax.broadcasted_iota(jnp.int32, (_M, _W), 1)
    u = (xs.astype(jnp.float32) - a.astype(jnp.float32) + 0.5) * (
        jnp.float32(_M) / ww) - 0.5
    u = jnp.clip(u, 0.0, jnp.float32(_M - 1))
    u0 = jnp.floor(u).astype(jnp.int32)
    u1 = jnp.minimum(u0 + 1, _M - 1)
    fu = u - u0.astype(jnp.float32)
    vx = (xs >= jnp.maximum(a, 0)) & (xs < jnp.minimum(c + 1, _W))
    r_u = ((kk == u0).astype(jnp.float32) * (1.0 - fu)
           + (kk == u1).astype(jnp.float32) * fu) * vx.astype(jnp.float32)

    tm = jax.lax.dot_general(
        r_vt, m, dimension_numbers=(((0,), (0,)), ((), ())),
        preferred_element_type=jnp.float32,
        precision=jax.lax.Precision.DEFAULT)
    tm = jnp.dot(tm, r_u, preferred_element_type=jnp.float32,
                 precision=jax.lax.Precision.DEFAULT)

    ys2 = jax.lax.broadcasted_iota(jnp.int32, (_H, 1), 0)
    xs2 = jax.lax.broadcasted_iota(jnp.int32, (1, _W), 1)
    rowm = ((ys2 >= cy1) & (ys2 < cy2)).astype(jnp.float32)
    colm = ((xs2 >= cx1) & (xs2 < cx2)).astype(jnp.float32)
    return tm + sem * rowm * colm


def _pan_kernel(*refs):
    sem_idx_ref, ibox_ref, fwh_ref = refs[:3]
    m_refs = refs[3:3 + _K]
    s_refs = refs[3 + _K:3 + 2 * _K]
    out_ref = refs[3 + 2 * _K]
    i = pl.program_id(0)
    for j, (m_ref, s_ref) in enumerate(zip(m_refs, s_refs)):
        ch = i * _K + j
        out_ref[j] = _paste_one(ibox_ref, fwh_ref, m_ref[0], s_ref[0], ch)


def _mask_spec(j):
    return pl.BlockSpec(
        (1, _M, _M),
        lambda i, sem_idx, ibox, fwh:
        (jnp.clip(i * _K + j - _STUFF_C, 0, _N - 1), 0, 0))


def _sem_spec(j):
    return pl.BlockSpec(
        (1, _H, _W),
        lambda i, sem_idx, ibox, fwh: (sem_idx[i * _K + j], 0, 0))


def kernel(mask_logit, sem_seg_logits, boxes, classes):
    bf = boxes / float(_STRIDE)
    bb = jnp.floor(bf).astype(jnp.int32)
    x0, y0, x2, y2 = bb[:, 0], bb[:, 1], bb[:, 2], bb[:, 3]
    w = (x2 - x0 + 1).astype(jnp.float32)
    h = (y2 - y0 + 1).astype(jnp.float32)
    cx1 = jnp.floor(bf[:, 0]).astype(jnp.int32)
    cy1 = jnp.floor(bf[:, 1]).astype(jnp.int32)
    cx2 = (jnp.round(bf[:, 2]) + 1.0).astype(jnp.int32)
    cy2 = (jnp.round(bf[:, 3]) + 1.0).astype(jnp.int32)
    ibox = jnp.stack([x0, y0, x2, y2, cx1, cy1, cx2, cy2], axis=1)
    fwh = jnp.stack([w, h], axis=1)
    stuff_ibox = jnp.tile(
        jnp.array([[0, 0, -2, -2, 0, 0, _W, _H]], dtype=jnp.int32),
        (_STUFF_C, 1))
    stuff_fwh = jnp.ones((_STUFF_C, 2), dtype=jnp.float32)
    ibox = jnp.concatenate([stuff_ibox, ibox])
    fwh = jnp.concatenate([stuff_fwh, fwh])
    sem_idx = jnp.concatenate([
        jnp.arange(_STUFF_C, dtype=jnp.int32),
        _STUFF_C + classes.astype(jnp.int32),
    ])

    grid_spec = pltpu.PrefetchScalarGridSpec(
        num_scalar_prefetch=3,
        grid=((_STUFF_C + _N) // _K,),
        in_specs=([_mask_spec(j) for j in range(_K)]
                  + [_sem_spec(j) for j in range(_K)]),
        out_specs=pl.BlockSpec((_K, _H, _W),
                               lambda i, sem_idx, ibox, fwh: (i, 0, 0)),
    )
    out = pl.pallas_call(
        _pan_kernel,
        grid_spec=grid_spec,
        out_shape=jax.ShapeDtypeStruct((_STUFF_C + _N, _H, _W), jnp.float32),
        compiler_params=pltpu.CompilerParams(
            dimension_semantics=("arbitrary",)),
    )(sem_idx, ibox, fwh, *([mask_logit] * _K), *([sem_seg_logits] * _K))
    return out[None]


# speedup vs baseline: 1.4556x; 1.0604x over previous
import jax
import jax.numpy as jnp
from jax.experimental import pallas as pl
from jax.experimental.pallas import tpu as pltpu

_N = 100
_M = 28
_H = 200
_W = 320
_STUFF_C = 53
_STRIDE = 4
_K = 17


def _paste_one(ibox_ref, fwh_ref, m, sem, n):
    a = ibox_ref[n, 0]
    b = ibox_ref[n, 1]
    c = ibox_ref[n, 2]
    d = ibox_ref[n, 3]
    cx1 = ibox_ref[n, 4]
    cy1 = ibox_ref[n, 5]
    cx2 = ibox_ref[n, 6]
    cy2 = ibox_ref[n, 7]
    ww = fwh_ref[n, 0]
    hh = fwh_ref[n, 1]

    jj = jax.lax.broadcasted_iota(jnp.int32, (_M, _H), 0)
    ys = jax.lax.broadcasted_iota(jnp.int32, (_M, _H), 1)
    v = (ys.astype(jnp.float32) - b.astype(jnp.float32) + 0.5) * (
        jnp.float32(_M) / hh) - 0.5
    v = jnp.clip(v, 0.0, jnp.float32(_M - 1))
    v0 = jnp.floor(v).astype(jnp.int32)
    v1 = jnp.minimum(v0 + 1, _M - 1)
    fv = v - v0.astype(jnp.float32)
    vy = (ys >= jnp.maximum(b, 0)) & (ys < jnp.minimum(d + 1, _H))
    r_vt = ((jj == v0).astype(jnp.float32) * (1.0 - fv)
            + (jj == v1).astype(jnp.float32) * fv) * vy.astype(jnp.float32)

    kk = jax.lax.broadcasted_iota(jnp.int32, (_M, _W), 0)
    xs = jax.lax.broadcasted_iota(jnp.int32, (_M, _W), 1)
    u = (xs.astype(jnp.float32) - a.astype(jnp.float32) + 0.5) * (
        jnp.float32(_M) / ww) - 0.5
    u = jnp.clip(u, 0.0, jnp.float32(_M - 1))
    u0 = jnp.floor(u).astype(jnp.int32)
    u1 = jnp.minimum(u0 + 1, _M - 1)
    fu = u - u0.astype(jnp.float32)
    vx = (xs >= jnp.maximum(a, 0)) & (xs < jnp.minimum(c + 1, _W))
    r_u = ((kk == u0).astype(jnp.float32) * (1.0 - fu)
           + (kk == u1).astype(jnp.float32) * fu) * vx.astype(jnp.float32)

    tm = jax.lax.dot_general(
        r_vt, m, dimension_numbers=(((0,), (0,)), ((), ())),
        preferred_element_type=jnp.float32,
        precision=jax.lax.Precision.DEFAULT)
    tm = jnp.dot(tm, r_u, preferred_element_type=jnp.float32,
                 precision=jax.lax.Precision.DEFAULT)

    ys2 = jax.lax.broadcasted_iota(jnp.int32, (_H, 1), 0)
    xs2 = jax.lax.broadcasted_iota(jnp.int32, (1, _W), 1)
    rowm = ((ys2 >= cy1) & (ys2 < cy2)).astype(jnp.float32)
    colm = ((xs2 >= cx1) & (xs2 < cx2)).astype(jnp.float32)
    return tm + sem * rowm * colm


def _pan_kernel(*refs):
    sem_idx_ref, ibox_ref, fwh_ref = refs[:3]
    m_refs = refs[3:3 + _K]
    s_refs = refs[3 + _K:3 + 2 * _K]
    out_ref = refs[3 + 2 * _K]
    i = pl.program_id(0)
    for j, (m_ref, s_ref) in enumerate(zip(m_refs, s_refs)):
        ch = i * _K + j
        out_ref[j] = _paste_one(ibox_ref, fwh_ref, m_ref[0], s_ref[0], ch)


def _mask_spec(j):
    return pl.BlockSpec(
        (1, _M, _M),
        lambda i, sem_idx, ibox, fwh:
        (jnp.clip(i * _K + j - _STUFF_C, 0, _N - 1), 0, 0))


def _sem_spec(j):
    return pl.BlockSpec(
        (1, _H, _W),
        lambda i, sem_idx, ibox, fwh: (sem_idx[i * _K + j], 0, 0))


def kernel(mask_logit, sem_seg_logits, boxes, classes):
    bf = boxes / float(_STRIDE)
    bb = jnp.floor(bf).astype(jnp.int32)
    x0, y0, x2, y2 = bb[:, 0], bb[:, 1], bb[:, 2], bb[:, 3]
    w = (x2 - x0 + 1).astype(jnp.float32)
    h = (y2 - y0 + 1).astype(jnp.float32)
    cx1 = jnp.floor(bf[:, 0]).astype(jnp.int32)
    cy1 = jnp.floor(bf[:, 1]).astype(jnp.int32)
    cx2 = (jnp.round(bf[:, 2]) + 1.0).astype(jnp.int32)
    cy2 = (jnp.round(bf[:, 3]) + 1.0).astype(jnp.int32)
    ibox = jnp.stack([x0, y0, x2, y2, cx1, cy1, cx2, cy2], axis=1)
    fwh = jnp.stack([w, h], axis=1)
    stuff_ibox = jnp.tile(
        jnp.array([[0, 0, -2, -2, 0, 0, _W, _H]], dtype=jnp.int32),
        (_STUFF_C, 1))
    stuff_fwh = jnp.ones((_STUFF_C, 2), dtype=jnp.float32)
    ibox = jnp.concatenate([stuff_ibox, ibox])
    fwh = jnp.concatenate([stuff_fwh, fwh])
    sem_idx = jnp.concatenate([
        jnp.arange(_STUFF_C, dtype=jnp.int32),
        _STUFF_C + classes.astype(jnp.int32),
    ])

    grid_spec = pltpu.PrefetchScalarGridSpec(
        num_scalar_prefetch=3,
        grid=((_STUFF_C + _N) // _K,),
        in_specs=([_mask_spec(j) for j in range(_K)]
                  + [_sem_spec(j) for j in range(_K)]),
        out_specs=pl.BlockSpec((_K, _H, _W),
                               lambda i, sem_idx, ibox, fwh: (i, 0, 0)),
    )
    out = pl.pallas_call(
        _pan_kernel,
        grid_spec=grid_spec,
        out_shape=jax.ShapeDtypeStruct((_STUFF_C + _N, _H, _W), jnp.float32),
        compiler_params=pltpu.CompilerParams(
            dimension_semantics=("arbitrary",)),
    )(sem_idx, ibox, fwh, *([mask_logit] * _K), *([sem_seg_logits] * _K))
    return out[None]
